# baseline (speedup 1.0000x reference)
import functools

import jax
import jax.numpy as jnp
from jax import lax
from jax.experimental import pallas as pl
from jax.experimental.pallas import tpu as pltpu
from jax.experimental.pallas import tpu_sc as plsc

NC = 1
NS = 8
L = 16
NW = NC * NS

B = 16384
BPW = B // NW
NCHUNK = 4
CW = BPW // NCHUNK
TAB = 1001

_mesh = plsc.VectorSubcoreMesh(
    core_axis_name="c", subcore_axis_name="s", num_cores=NC, num_subcores=NS
)


@functools.partial(
    pl.kernel,
    mesh=_mesh,
    out_type=jax.ShapeDtypeStruct((B,), jnp.float32),
    scratch_types=[
        pltpu.VMEM((TAB,), jnp.float32),
        pltpu.VMEM((BPW,), jnp.int32),
        pltpu.VMEM((BPW,), jnp.float32),
        pltpu.SemaphoreType.DMA,
        pltpu.SemaphoreType.DMA,
        pltpu.SemaphoreType.DMA,
    ],
    compiler_params=pltpu.CompilerParams(
        needs_layout_passes=False,
        skip_device_barrier=True,
        disable_bounds_checks=True,
        disable_semaphore_checks=True,
    ),
)
def _gather_kernel(
    gam_hbm, t_hbm, out_hbm, gam_v, idx_v, out_v, sem_g, sem_t, sem_o
):
    wid = lax.axis_index("s") * NC + lax.axis_index("c")
    base = wid * BPW
    cp_g = pltpu.async_copy(gam_hbm, gam_v, sem_g)
    cp_t = pltpu.async_copy(t_hbm.at[pl.ds(base, BPW)], idx_v, sem_t)
    cp_g.wait()
    cp_t.wait()
    out_cps = []
    for c in range(NCHUNK):
        for j in range(CW // L):
            o = c * CW + j * L
            idx = idx_v[pl.ds(o, L)]
            out_v[pl.ds(o, L)] = plsc.load_gather(gam_v, [idx])
        out_cps.append(
            pltpu.async_copy(
                out_v.at[pl.ds(c * CW, CW)],
                out_hbm.at[pl.ds(base + c * CW, CW)],
                sem_o,
            )
        )
    for cp in out_cps:
        cp.wait()


def kernel(t, gammas):
    return _gather_kernel(gammas.astype(jnp.float32), t.astype(jnp.int32))


# speedup vs baseline: 1.0121x; 1.0121x over previous
import functools

import jax
import jax.numpy as jnp
from jax import lax
from jax.experimental import pallas as pl
from jax.experimental.pallas import tpu as pltpu
from jax.experimental.pallas import tpu_sc as plsc

NC = 1
NS = 16
L = 16
NW = NC * NS

B = 16384
BPW = B // NW
NCHUNK = 8
CW = BPW // NCHUNK
TAB = 1001

_mesh = plsc.VectorSubcoreMesh(
    core_axis_name="c", subcore_axis_name="s", num_cores=NC
)


@functools.partial(
    pl.kernel,
    mesh=_mesh,
    out_type=jax.ShapeDtypeStruct((B,), jnp.float32),
    scratch_types=[
        pltpu.VMEM((TAB,), jnp.float32),
        pltpu.VMEM((BPW,), jnp.int32),
        pltpu.VMEM((BPW,), jnp.float32),
        pltpu.SemaphoreType.DMA,
        pltpu.SemaphoreType.DMA,
        pltpu.SemaphoreType.DMA,
    ],
    compiler_params=pltpu.CompilerParams(
        needs_layout_passes=False,
        skip_device_barrier=True,
        disable_bounds_checks=True,
        disable_semaphore_checks=True,
    ),
)
def _gather_kernel(
    gam_hbm, t_hbm, out_hbm, gam_v, idx_v, out_v, sem_g, sem_t, sem_o
):
    wid = lax.axis_index("s") * NC + lax.axis_index("c")
    base = wid * BPW
    cp_g = pltpu.async_copy(gam_hbm, gam_v, sem_g)
    cp_t = pltpu.async_copy(t_hbm.at[pl.ds(base, BPW)], idx_v, sem_t)
    cp_g.wait()
    cp_t.wait()
    out_cps = []
    for c in range(NCHUNK):
        for j in range(CW // L):
            o = c * CW + j * L
            idx = idx_v[pl.ds(o, L)]
            out_v[pl.ds(o, L)] = plsc.load_gather(gam_v, [idx])
        out_cps.append(
            pltpu.async_copy(
                out_v.at[pl.ds(c * CW, CW)],
                out_hbm.at[pl.ds(base + c * CW, CW)],
                sem_o,
            )
        )
    for cp in out_cps:
        cp.wait()


def kernel(t, gammas):
    return _gather_kernel(gammas.astype(jnp.float32), t.astype(jnp.int32))


# speedup vs baseline: 1.0302x; 1.0179x over previous
import functools

import jax
import jax.numpy as jnp
from jax import lax
from jax.experimental import pallas as pl
from jax.experimental.pallas import tpu as pltpu
from jax.experimental.pallas import tpu_sc as plsc

NC = 1
NS = 16
L = 16
NW = NC * NS

B = 16384
BPW = B // NW
NCHUNK = 8
CW = BPW // NCHUNK
TAB = 1001

_mesh = plsc.VectorSubcoreMesh(
    core_axis_name="c", subcore_axis_name="s", num_cores=NC
)


@functools.partial(
    pl.kernel,
    mesh=_mesh,
    out_type=jax.ShapeDtypeStruct((B,), jnp.float32),
    scratch_types=[
        pltpu.VMEM((TAB,), jnp.float32),
        pltpu.VMEM((BPW,), jnp.int32),
        pltpu.VMEM((BPW,), jnp.float32),
        pltpu.SemaphoreType.DMA,
        pltpu.SemaphoreType.DMA,
        pltpu.SemaphoreType.DMA,
    ],
    compiler_params=pltpu.CompilerParams(
        needs_layout_passes=False,
        skip_device_barrier=True,
        disable_bounds_checks=True,
        disable_semaphore_checks=True,
    ),
)
def _gather_kernel(
    gam_hbm, t_hbm, out_hbm, gam_v, idx_v, out_v, sem_g, sem_t, sem_o
):
    wid = lax.axis_index("s") * NC + lax.axis_index("c")
    base = wid * BPW
    cp_g = pltpu.async_copy(gam_hbm, gam_v, sem_g)
    cp_t = pltpu.async_copy(t_hbm.at[pl.ds(base, BPW)], idx_v, sem_t)
    cp_g.wait()
    cp_t.wait()

    def body(i, _):
        for u in range(4):
            o = i * 4 * L + u * L
            idx = idx_v[pl.ds(o, L)]
            out_v[pl.ds(o, L)] = plsc.load_gather(gam_v, [idx])
        return 0

    lax.fori_loop(0, BPW // (4 * L), body, 0)
    pltpu.async_copy(out_v, out_hbm.at[pl.ds(base, BPW)], sem_o).wait()


def kernel(t, gammas):
    return _gather_kernel(gammas.astype(jnp.float32), t.astype(jnp.int32))


# speedup vs baseline: 1.0416x; 1.0111x over previous
import functools

import jax
import jax.numpy as jnp
from jax import lax
from jax.experimental import pallas as pl
from jax.experimental.pallas import tpu as pltpu
from jax.experimental.pallas import tpu_sc as plsc

NC = 1
NS = 16
L = 16
NW = NC * NS

B = 16384
BPW = B // NW
NCHUNK = 8
CW = BPW // NCHUNK
TAB = 1001

_mesh = plsc.VectorSubcoreMesh(
    core_axis_name="c", subcore_axis_name="s", num_cores=NC
)


@functools.partial(
    pl.kernel,
    mesh=_mesh,
    out_type=jax.ShapeDtypeStruct((B,), jnp.float32),
    scratch_types=[
        pltpu.VMEM((TAB,), jnp.float32),
        pltpu.VMEM((BPW,), jnp.int32),
        pltpu.VMEM((BPW,), jnp.float32),
        pltpu.SemaphoreType.DMA,
        pltpu.SemaphoreType.DMA,
        pltpu.SemaphoreType.DMA,
    ],
    compiler_params=pltpu.CompilerParams(
        needs_layout_passes=False,
        skip_device_barrier=True,
        disable_bounds_checks=True,
        disable_semaphore_checks=True,
    ),
)
def _gather_kernel(
    gam_hbm, t_hbm, out_hbm, gam_v, idx_v, out_v, sem_g, sem_t, sem_o
):
    wid = lax.axis_index("s") * NC + lax.axis_index("c")
    base = wid * BPW
    cp_g = pltpu.async_copy(gam_hbm, gam_v, sem_g)
    cp_t = pltpu.async_copy(t_hbm.at[pl.ds(base, BPW)], idx_v, sem_t)
    cp_g.wait()
    cp_t.wait()

    @plsc.parallel_loop(0, BPW // L, unroll=4)
    def body(i):
        o = i * L
        idx = idx_v[pl.ds(o, L)]
        out_v[pl.ds(o, L)] = plsc.load_gather(gam_v, [idx])
    pltpu.async_copy(out_v, out_hbm.at[pl.ds(base, BPW)], sem_o).wait()


def kernel(t, gammas):
    return _gather_kernel(gammas.astype(jnp.float32), t.astype(jnp.int32))
